# baseline (speedup 1.0000x reference)
import functools

import jax
import jax.numpy as jnp
from jax import lax
from jax.experimental import pallas as pl
from jax.experimental.pallas import tpu as pltpu
from jax.experimental.pallas import tpu_sc as plsc

V = 1000000
D = 64
B = 16384
NS = 16
BPT = B // NS
DPC = D // 2
VB = V - V % 128
CH = 125056
LAST = VB - 7 * CH
VP = V + 128 - V % 128
TS = VP - 1024


@functools.cache
def _build():
    mesh = plsc.VectorSubcoreMesh(core_axis_name="c", subcore_axis_name="s")

    @functools.partial(
        pl.kernel,
        out_type=jax.ShapeDtypeStruct((D * NS, BPT), jnp.float32),
        mesh=mesh,
        scratch_types=[
            pltpu.VMEM_SHARED((VP,), jnp.float32),
            pltpu.VMEM_SHARED((VP,), jnp.float32),
            pltpu.VMEM((2 * BPT,), jnp.int32),
            pltpu.VMEM((BPT,), jnp.float32),
            pltpu.VMEM((BPT,), jnp.float32),
            pltpu.SemaphoreType.DMA,
            pltpu.SemaphoreType.DMA,
        ],
    )
    def _emb_prod(idxall_hbm, at_hbm, bt_hbm, tail_hbm, out_hbm,
                  shA, shB, idxall_v, aval_v, bval_v, sem, gsem):
        cid = lax.axis_index("c")
        sid = lax.axis_index("s")
        pltpu.sync_copy(idxall_hbm.at[sid], idxall_v)
        idx0_v = idxall_v.at[pl.ds(0, BPT)]
        idx1_v = idxall_v.at[pl.ds(BPT, BPT)]

        off = (sid % 8) * CH
        d0 = cid * DPC

        def copies(d):
            c7 = [(at_hbm.at[d].at[pl.ds(7 * CH, LAST)],
                   shA.at[pl.ds(7 * CH, LAST)]),
                  (tail_hbm.at[d], shA.at[pl.ds(TS, 1024)])]
            c15 = [(bt_hbm.at[d].at[pl.ds(7 * CH, LAST)],
                    shB.at[pl.ds(7 * CH, LAST)]),
                   (tail_hbm.at[D + d], shB.at[pl.ds(TS, 1024)])]
            ca = [(at_hbm.at[d].at[pl.ds(off, CH)], shA.at[pl.ds(off, CH)])]
            cb = [(bt_hbm.at[d].at[pl.ds(off, CH)], shB.at[pl.ds(off, CH)])]
            return c7, c15, ca, cb

        def fire_loads(d):
            c7, c15, ca, cb = copies(d)

            @pl.when(sid < 7)
            def _():
                for s, t in ca:
                    pltpu.async_copy(s, t, sem)

            @pl.when(sid == 7)
            def _():
                for s, t in c7:
                    pltpu.async_copy(s, t, sem)

            @pl.when((sid >= 8) & (sid < 15))
            def _():
                for s, t in cb:
                    pltpu.async_copy(s, t, sem)

            @pl.when(sid == 15)
            def _():
                for s, t in c15:
                    pltpu.async_copy(s, t, sem)

        def drain_loads(d):
            c7, c15, ca, cb = copies(d)

            @pl.when(sid < 7)
            def _():
                for s, t in ca:
                    pltpu.make_async_copy(s, t, sem).wait()

            @pl.when(sid == 7)
            def _():
                for s, t in c7:
                    pltpu.make_async_copy(s, t, sem).wait()

            @pl.when((sid >= 8) & (sid < 15))
            def _():
                for s, t in cb:
                    pltpu.make_async_copy(s, t, sem).wait()

            @pl.when(sid == 15)
            def _():
                for s, t in c15:
                    pltpu.make_async_copy(s, t, sem).wait()

        def mult_and_store(i):
            def mbody(k, c2):
                sl = pl.ds(k * 16, 16)
                aval_v[sl] = aval_v[sl] * bval_v[sl]
                return c2
            lax.fori_loop(0, BPT // 16, mbody, 0)
            pltpu.sync_copy(aval_v, out_hbm.at[(d0 + i) * NS + sid])

        fire_loads(d0)
        for i in range(DPC):
            drain_loads(d0 + i)
            plsc.subcore_barrier()
            ga = pltpu.async_copy(shA.at[idx0_v], aval_v, gsem)
            gb = pltpu.async_copy(shB.at[idx1_v], bval_v, gsem)
            ga.wait()
            gb.wait()
            plsc.subcore_barrier()
            if i + 1 < DPC:
                fire_loads(d0 + i + 1)
            mult_and_store(i)

    return _emb_prod


def kernel(g, emb_in, emb_out):
    g = g.astype(jnp.int32)
    idx_all = jnp.concatenate(
        [g[:, 0].reshape(NS, BPT), g[:, 1].reshape(NS, BPT)], axis=1)
    pad = ((0, VP - V), (0, 0))
    tail = jnp.concatenate(
        [jnp.pad(emb_in[TS:], pad).T, jnp.pad(emb_out[TS:], pad).T], axis=0)
    out_p = _build()(idx_all, emb_in.T, emb_out.T, tail)
    return out_p.reshape(D, B).T


# speedup vs baseline: 1.0050x; 1.0050x over previous
import functools

import jax
import jax.numpy as jnp
from jax import lax
from jax.experimental import pallas as pl
from jax.experimental.pallas import tpu as pltpu
from jax.experimental.pallas import tpu_sc as plsc

V = 1000000
D = 64
B = 16384
NS = 16
BPT = B // NS
DPC = D // 2
VB = V - V % 128
CH = 125056
LAST = VB - 7 * CH
VP = V + 128 - V % 128
TS = VP - 1024


@functools.cache
def _build():
    mesh = plsc.VectorSubcoreMesh(core_axis_name="c", subcore_axis_name="s")

    @functools.partial(
        pl.kernel,
        out_type=jax.ShapeDtypeStruct((D * NS, BPT), jnp.float32),
        mesh=mesh,
        scratch_types=[
            pltpu.VMEM_SHARED((VP,), jnp.float32),
            pltpu.VMEM_SHARED((VP,), jnp.float32),
            pltpu.VMEM((2 * BPT,), jnp.int32),
            pltpu.VMEM((BPT,), jnp.float32),
            pltpu.VMEM((BPT,), jnp.float32),
            pltpu.SemaphoreType.DMA,
            pltpu.SemaphoreType.DMA,
        ],
    )
    def _emb_prod(idxall_hbm, at_hbm, bt_hbm, tail_hbm, out_hbm,
                  shA, shB, idxall_v, aval_v, bval_v, sem, gsem):
        cid = lax.axis_index("c")
        sid = lax.axis_index("s")
        pltpu.sync_copy(idxall_hbm.at[sid], idxall_v)
        idx0_v = idxall_v.at[pl.ds(0, BPT)]
        idx1_v = idxall_v.at[pl.ds(BPT, BPT)]

        off = (sid % 8) * CH

        def copies(d):
            c7 = [(at_hbm.at[d].at[pl.ds(7 * CH, LAST)],
                   shA.at[pl.ds(7 * CH, LAST)]),
                  (tail_hbm.at[d], shA.at[pl.ds(TS, 1024)])]
            c15 = [(bt_hbm.at[d].at[pl.ds(7 * CH, LAST)],
                    shB.at[pl.ds(7 * CH, LAST)]),
                   (tail_hbm.at[D + d], shB.at[pl.ds(TS, 1024)])]
            ca = [(at_hbm.at[d].at[pl.ds(off, CH)], shA.at[pl.ds(off, CH)])]
            cb = [(bt_hbm.at[d].at[pl.ds(off, CH)], shB.at[pl.ds(off, CH)])]
            return c7, c15, ca, cb

        def fire_loads(d):
            c7, c15, ca, cb = copies(d)

            @pl.when(sid < 7)
            def _():
                for s, t in ca:
                    pltpu.async_copy(s, t, sem)

            @pl.when(sid == 7)
            def _():
                for s, t in c7:
                    pltpu.async_copy(s, t, sem)

            @pl.when((sid >= 8) & (sid < 15))
            def _():
                for s, t in cb:
                    pltpu.async_copy(s, t, sem)

            @pl.when(sid == 15)
            def _():
                for s, t in c15:
                    pltpu.async_copy(s, t, sem)

        def drain_loads(d):
            c7, c15, ca, cb = copies(d)

            @pl.when(sid < 7)
            def _():
                for s, t in ca:
                    pltpu.make_async_copy(s, t, sem).wait()

            @pl.when(sid == 7)
            def _():
                for s, t in c7:
                    pltpu.make_async_copy(s, t, sem).wait()

            @pl.when((sid >= 8) & (sid < 15))
            def _():
                for s, t in cb:
                    pltpu.make_async_copy(s, t, sem).wait()

            @pl.when(sid == 15)
            def _():
                for s, t in c15:
                    pltpu.make_async_copy(s, t, sem).wait()

        def mult_and_store(i):
            def mbody(k, c2):
                sl = pl.ds(k * 16, 16)
                aval_v[sl] = aval_v[sl] * bval_v[sl]
                return c2
            lax.fori_loop(0, BPT // 16, mbody, 0)
            pltpu.sync_copy(aval_v, out_hbm.at[(2 * i + cid) * NS + sid])

        fire_loads(cid)
        for i in range(DPC):
            drain_loads(2 * i + cid)
            plsc.subcore_barrier()
            ga = pltpu.async_copy(shA.at[idx0_v], aval_v, gsem)
            gb = pltpu.async_copy(shB.at[idx1_v], bval_v, gsem)
            ga.wait()
            gb.wait()
            plsc.subcore_barrier()
            if i + 1 < DPC:
                fire_loads(2 * (i + 1) + cid)
            mult_and_store(i)

    return _emb_prod


def kernel(g, emb_in, emb_out):
    g = g.astype(jnp.int32)
    idx_all = jnp.concatenate(
        [g[:, 0].reshape(NS, BPT), g[:, 1].reshape(NS, BPT)], axis=1)
    pad = ((0, VP - V), (0, 0))
    tail = jnp.concatenate(
        [jnp.pad(emb_in[TS:], pad).T, jnp.pad(emb_out[TS:], pad).T], axis=0)
    out_p = _build()(idx_all, emb_in.T, emb_out.T, tail)
    return out_p.reshape(D, B).T
